# baseline (speedup 1.0000x reference)
import jax
import jax.numpy as jnp
from jax import lax
from jax.experimental import pallas as pl
from jax.experimental.pallas import tpu as pltpu

VOCAB = 100000
EMB = 64
CTX = 200
FLAT = CTX * EMB

_KBLK = 1280
_KSTEPS = FLAT // _KBLK
_VSTEPS = 10
_VBLK = VOCAB // _VSTEPS
_GRID = _KSTEPS + _VSTEPS


def _gather_body(idx_ref, emb_ref, out_ref, sem):
    def start(t, carry):
        pltpu.make_async_copy(
            emb_ref.at[pl.ds(idx_ref[t], 1), :],
            out_ref.at[pl.ds(t, 1), :],
            sem).start()
        return carry

    lax.fori_loop(0, CTX, start, 0)

    def drain(t, carry):
        pltpu.make_async_copy(
            emb_ref.at[pl.ds(0, 1), :],
            out_ref.at[pl.ds(0, 1), :],
            sem).wait()
        return carry

    lax.fori_loop(0, CTX, drain, 0)


def _tc_gather(emb, idx):
    return pl.pallas_call(
        _gather_body,
        in_specs=[
            pl.BlockSpec(memory_space=pltpu.SMEM),
            pl.BlockSpec(memory_space=pltpu.MemorySpace.HBM),
        ],
        out_specs=pl.BlockSpec(memory_space=pltpu.VMEM),
        out_shape=jax.ShapeDtypeStruct((CTX, EMB), jnp.float32),
        scratch_shapes=[pltpu.SemaphoreType.DMA],
    )(idx, emb)


def _dense_body(x_ref, waa_ref, wab_ref, ba_ref, w1_ref, b1_ref, w2_ref,
                b2_ref, w3a_ref, w3b_ref, b3_ref, out_ref, acc_ref, h2_ref):
    j = pl.program_id(0)

    @pl.when(j == 0)
    def _():
        acc_ref[...] = jnp.zeros_like(acc_ref)

    @pl.when(j < _KSTEPS)
    def _():
        acc_ref[:, pl.ds(0, 256)] += lax.dot_general(
            x_ref[...], waa_ref[...], (((1,), (1,)), ((), ())),
            preferred_element_type=jnp.float32)
        acc_ref[:, pl.ds(256, 256)] += lax.dot_general(
            x_ref[...], wab_ref[...], (((1,), (1,)), ((), ())),
            preferred_element_type=jnp.float32)

    @pl.when(j == _KSTEPS - 1)
    def _():
        h0 = acc_ref[...] + ba_ref[...]
        h1 = jax.nn.relu(
            lax.dot_general(h0, w1_ref[...], (((1,), (1,)), ((), ())),
                            preferred_element_type=jnp.float32) + b1_ref[...])
        h2_ref[...] = jax.nn.relu(
            lax.dot_general(h1, w2_ref[...], (((1,), (1,)), ((), ())),
                            preferred_element_type=jnp.float32) + b2_ref[...])

    @pl.when(j >= _KSTEPS)
    def _():
        v = j - _KSTEPS
        row_a = lax.dot_general(
            h2_ref[...], w3a_ref[...], (((1,), (1,)), ((), ())),
            preferred_element_type=jnp.float32)
        row_b = lax.dot_general(
            h2_ref[...], w3b_ref[...], (((1,), (1,)), ((), ())),
            preferred_element_type=jnp.float32)
        half = _VBLK // 2
        out_ref[pl.ds(v, 1), pl.ds(0, half)] = (
            row_a + b3_ref[pl.ds(v, 1), pl.ds(0, half)])
        out_ref[pl.ds(v, 1), pl.ds(half, half)] = (
            row_b + b3_ref[pl.ds(v, 1), pl.ds(half, half)])

    @pl.when(j == _GRID - 1)
    def _():
        logits = out_ref[...]
        m = jnp.max(logits)
        lse = m + jnp.log(jnp.sum(jnp.exp(logits - m)))
        out_ref[...] = logits - lse


def _dense_stack(x, W_aug, b_aug, W1, b1, W2, b2, W3, b3):
    return pl.pallas_call(
        _dense_body,
        grid=(_GRID,),
        in_specs=[
            pl.BlockSpec((1, _KBLK), lambda j: (0, jnp.minimum(j, _KSTEPS - 1))),
            pl.BlockSpec((256, _KBLK),
                         lambda j: (0, jnp.minimum(j, _KSTEPS - 1))),
            pl.BlockSpec((256, _KBLK),
                         lambda j: (1, jnp.minimum(j, _KSTEPS - 1))),
            pl.BlockSpec((1, 512), lambda j: (0, 0)),
            pl.BlockSpec((128, 512), lambda j: (0, 0)),
            pl.BlockSpec((1, 128), lambda j: (0, 0)),
            pl.BlockSpec((64, 128), lambda j: (0, 0)),
            pl.BlockSpec((1, 64), lambda j: (0, 0)),
            pl.BlockSpec((_VBLK // 2, 64),
                         lambda j: (2 * jnp.maximum(j - _KSTEPS, 0), 0)),
            pl.BlockSpec((_VBLK // 2, 64),
                         lambda j: (2 * jnp.maximum(j - _KSTEPS, 0) + 1, 0)),
            pl.BlockSpec((_VSTEPS, _VBLK), lambda j: (0, 0)),
        ],
        out_specs=pl.BlockSpec((_VSTEPS, _VBLK), lambda j: (0, 0)),
        out_shape=jax.ShapeDtypeStruct((_VSTEPS, _VBLK), jnp.float32),
        scratch_shapes=[
            pltpu.VMEM((1, 512), jnp.float32),
            pltpu.VMEM((1, 64), jnp.float32),
        ],
    )(x, W_aug, W_aug, b_aug.reshape(1, 512), W1, b1.reshape(1, 128),
      W2, b2.reshape(1, 64), W3, W3, b3.reshape(_VSTEPS, _VBLK))


def kernel(inputs, emb, W_aug, b_aug, W1, b1, W2, b2, W3, b3):
    idx = inputs.astype(jnp.int32)
    rows = _tc_gather(emb, idx)
    x = rows.reshape(1, FLAT)
    out = _dense_stack(x, W_aug, b_aug, W1, b1, W2, b2, W3, b3)
    return out.reshape(1, VOCAB)


# speedup vs baseline: 2.3499x; 2.3499x over previous
import jax
import jax.numpy as jnp
from jax import lax
from jax.experimental import pallas as pl
from jax.experimental.pallas import tpu as pltpu

VOCAB = 100000
EMB = 64
CTX = 200
FLAT = CTX * EMB

_KBLK = 1280
_KSTEPS = FLAT // _KBLK
_ROWS_PER_K = _KBLK // EMB
_VROWS = 8
_VSTEPS = EMB // _VROWS
_GRID = _KSTEPS + _VSTEPS


def _body(idx_ref, embT_ref, wa_ref, ba_ref, w1_ref, b1_ref, w2t_ref,
          b2_ref, w3t_ref, b3_ref, out_ref, bcols_ref, xcol_ref, acc_ref,
          h2_ref, sems):
    j = pl.program_id(0)

    @pl.when(j == 0)
    def _():
        acc_ref[...] = jnp.zeros_like(acc_ref)
        for t in range(CTX):
            base = pl.multiple_of((idx_ref[t] // 128) * 128, 128)
            pltpu.make_async_copy(
                embT_ref.at[:, pl.ds(base, 128)],
                bcols_ref.at[:, pl.ds(t * 128, 128)],
                sems.at[t // _ROWS_PER_K]).start()

    @pl.when(j < _KSTEPS)
    def _():
        for _ in range(_ROWS_PER_K):
            pltpu.make_async_copy(
                embT_ref.at[:, pl.ds(0, 128)],
                bcols_ref.at[:, pl.ds(0, 128)],
                sems.at[j]).wait()

        lane_iota = lax.broadcasted_iota(jnp.int32, (1, 128), 1)

        def extract(i, carry):
            t = j * _ROWS_PER_K + i
            lane = idx_ref[t] % 128
            off = pl.multiple_of(t * 128, 128)
            blk = bcols_ref[:, pl.ds(off, 128)]
            oh = (lane_iota == lane).astype(jnp.float32)
            v = jnp.sum(blk * oh, axis=1, keepdims=True)
            xcol_ref[pl.ds(t * EMB, EMB), :] = v
            return carry

        lax.fori_loop(0, _ROWS_PER_K, extract, 0)

        acc_ref[...] += lax.dot_general(
            wa_ref[...], xcol_ref[pl.ds(j * _KBLK, _KBLK), :],
            (((1,), (0,)), ((), ())), preferred_element_type=jnp.float32)

    @pl.when(j == _KSTEPS - 1)
    def _():
        h0 = acc_ref[...] + ba_ref[...]
        h1 = jax.nn.relu(
            lax.dot_general(w1_ref[...], h0, (((1,), (0,)), ((), ())),
                            preferred_element_type=jnp.float32) + b1_ref[...])
        h2row = jax.nn.relu(
            lax.dot_general(h1, w2t_ref[...], (((0,), (0,)), ((), ())),
                            preferred_element_type=jnp.float32) + b2_ref[...])
        for i in range(_VSTEPS):
            h2_ref[i:i + 1, :] = h2row[:, i * _VROWS:(i + 1) * _VROWS]

    @pl.when(j >= _KSTEPS)
    def _():
        k = j - _KSTEPS
        part = lax.dot_general(
            h2_ref[pl.ds(k, 1), :], w3t_ref[...],
            (((1,), (0,)), ((), ())), preferred_element_type=jnp.float32)

        @pl.when(k == 0)
        def _():
            out_ref[...] = part + b3_ref[...]

        @pl.when(k > 0)
        def _():
            out_ref[...] += part

    @pl.when(j == _GRID - 1)
    def _():
        logits = out_ref[...]
        m = jnp.max(logits)
        lse = m + jnp.log(jnp.sum(jnp.exp(logits - m)))
        out_ref[...] = logits - lse


def kernel(inputs, emb, W_aug, b_aug, W1, b1, W2, b2, W3, b3):
    idx = inputs.astype(jnp.int32)
    return pl.pallas_call(
        _body,
        grid=(_GRID,),
        in_specs=[
            pl.BlockSpec(memory_space=pltpu.SMEM),
            pl.BlockSpec(memory_space=pltpu.MemorySpace.HBM),
            pl.BlockSpec((512, _KBLK), lambda j: (0, jnp.minimum(j, _KSTEPS - 1))),
            pl.BlockSpec((512, 1), lambda j: (0, 0)),
            pl.BlockSpec((128, 512), lambda j: (0, 0)),
            pl.BlockSpec((128, 1), lambda j: (0, 0)),
            pl.BlockSpec((128, 64), lambda j: (0, 0)),
            pl.BlockSpec((1, 64), lambda j: (0, 0)),
            pl.BlockSpec((_VROWS, VOCAB),
                         lambda j: (jnp.maximum(j - _KSTEPS, 0), 0)),
            pl.BlockSpec((1, VOCAB), lambda j: (0, 0)),
        ],
        out_specs=pl.BlockSpec((1, VOCAB), lambda j: (0, 0)),
        out_shape=jax.ShapeDtypeStruct((1, VOCAB), jnp.float32),
        scratch_shapes=[
            pltpu.VMEM((EMB, CTX * 128), jnp.float32),
            pltpu.VMEM((FLAT, 1), jnp.float32),
            pltpu.VMEM((512, 1), jnp.float32),
            pltpu.VMEM((_VSTEPS, _VROWS), jnp.float32),
            pltpu.SemaphoreType.DMA((_KSTEPS,)),
        ],
    )(idx, emb.T, W_aug, b_aug.reshape(512, 1), W1, b1.reshape(128, 1),
      W2.T, b2.reshape(1, 64), W3.T, b3.reshape(1, VOCAB))


# speedup vs baseline: 2.6557x; 1.1301x over previous
import jax
import jax.numpy as jnp
from jax import lax
from jax.experimental import pallas as pl
from jax.experimental.pallas import tpu as pltpu

VOCAB = 100000
EMB = 64
CTX = 200
FLAT = CTX * EMB

_KBLK = 1280
_KSTEPS = FLAT // _KBLK
_ROWS_PER_K = _KBLK // EMB
_GRID = _KSTEPS + 1


def _body(idx_ref, embT_ref, w3t_hbm, wa_ref, ba_ref, w1_ref, b1_ref,
          w2t_ref, b2_ref, b3_ref, out_ref, bcols_ref, xcol_ref, acc_ref,
          h2_ref, w3t_vmem, sems, w3sem):
    j = pl.program_id(0)

    @pl.when(j == 0)
    def _():
        acc_ref[...] = jnp.zeros_like(acc_ref)
        for t in range(CTX):
            base = pl.multiple_of((idx_ref[t] // 128) * 128, 128)
            pltpu.make_async_copy(
                embT_ref.at[:, pl.ds(base, 128)],
                bcols_ref.at[:, pl.ds(t * 128, 128)],
                sems.at[t // _ROWS_PER_K]).start()
        pltpu.make_async_copy(w3t_hbm, w3t_vmem, w3sem).start()

    @pl.when(j < _KSTEPS)
    def _():
        for _ in range(_ROWS_PER_K):
            pltpu.make_async_copy(
                embT_ref.at[:, pl.ds(0, 128)],
                bcols_ref.at[:, pl.ds(0, 128)],
                sems.at[j]).wait()

        lane_iota = lax.broadcasted_iota(jnp.int32, (1, 128), 1)

        def extract(i, carry):
            t = j * _ROWS_PER_K + i
            lane = idx_ref[t] % 128
            off = pl.multiple_of(t * 128, 128)
            blk = bcols_ref[:, pl.ds(off, 128)]
            oh = (lane_iota == lane).astype(jnp.float32)
            v = jnp.sum(blk * oh, axis=1, keepdims=True)
            xcol_ref[pl.ds(t * EMB, EMB), :] = v
            return carry

        lax.fori_loop(0, _ROWS_PER_K, extract, 0)

        acc_ref[...] += lax.dot_general(
            wa_ref[...], xcol_ref[pl.ds(j * _KBLK, _KBLK), :],
            (((1,), (0,)), ((), ())), preferred_element_type=jnp.float32)

    @pl.when(j == _KSTEPS - 1)
    def _():
        h0 = acc_ref[...] + ba_ref[...]
        h1 = jax.nn.relu(
            lax.dot_general(w1_ref[...], h0, (((1,), (0,)), ((), ())),
                            preferred_element_type=jnp.float32) + b1_ref[...])
        h2_ref[...] = jax.nn.relu(
            lax.dot_general(h1, w2t_ref[...], (((0,), (0,)), ((), ())),
                            preferred_element_type=jnp.float32) + b2_ref[...])

    @pl.when(j == _KSTEPS)
    def _():
        pltpu.make_async_copy(w3t_hbm, w3t_vmem, w3sem).wait()
        logits = lax.dot_general(
            h2_ref[...], w3t_vmem[...], (((1,), (0,)), ((), ())),
            preferred_element_type=jnp.float32) + b3_ref[...]
        m = jnp.max(logits)
        lse = m + jnp.log(jnp.sum(jnp.exp(logits - m)))
        out_ref[...] = logits - lse


def kernel(inputs, emb, W_aug, b_aug, W1, b1, W2, b2, W3, b3):
    idx = inputs.astype(jnp.int32)
    return pl.pallas_call(
        _body,
        grid=(_GRID,),
        in_specs=[
            pl.BlockSpec(memory_space=pltpu.SMEM),
            pl.BlockSpec(memory_space=pltpu.MemorySpace.HBM),
            pl.BlockSpec(memory_space=pltpu.MemorySpace.HBM),
            pl.BlockSpec((512, _KBLK), lambda j: (0, jnp.minimum(j, _KSTEPS - 1))),
            pl.BlockSpec((512, 1), lambda j: (0, 0)),
            pl.BlockSpec((128, 512), lambda j: (0, 0)),
            pl.BlockSpec((128, 1), lambda j: (0, 0)),
            pl.BlockSpec((128, 64), lambda j: (0, 0)),
            pl.BlockSpec((1, 64), lambda j: (0, 0)),
            pl.BlockSpec((1, VOCAB), lambda j: (0, 0)),
        ],
        out_specs=pl.BlockSpec((1, VOCAB), lambda j: (0, 0)),
        out_shape=jax.ShapeDtypeStruct((1, VOCAB), jnp.float32),
        scratch_shapes=[
            pltpu.VMEM((EMB, CTX * 128), jnp.float32),
            pltpu.VMEM((FLAT, 1), jnp.float32),
            pltpu.VMEM((512, 1), jnp.float32),
            pltpu.VMEM((1, EMB), jnp.float32),
            pltpu.VMEM((EMB, VOCAB), jnp.float32),
            pltpu.SemaphoreType.DMA((_KSTEPS,)),
            pltpu.SemaphoreType.DMA,
        ],
        compiler_params=pltpu.CompilerParams(
            vmem_limit_bytes=100 * 1024 * 1024),
    )(idx, emb.T, W3.T, W_aug, b_aug.reshape(512, 1), W1,
      b1.reshape(128, 1), W2.T, b2.reshape(1, 64), b3.reshape(1, VOCAB))
